# baseline (speedup 1.0000x reference)
import functools

import jax
import jax.numpy as jnp
from jax import lax
from jax.experimental import pallas as pl
from jax.experimental.pallas import tpu as pltpu
from jax.experimental.pallas import tpu_sc as plsc

NUM_ACTIONS = 1000000
LATENT_DIM = 32
BATCH = 16384
HIST = 50

_NC = 2
_NS = 16
_NW = _NC * _NS

_B = BATCH * HIST
_BPW = _B // _NW
_CHUNK = 1600
_NROWS = _CHUNK // HIST
_NCHUNK = _BPW // _CHUNK


def _gather_kernel(table_hbm, idx_hbm, out_hbm,
                   idx0, idx1, rows0, rows1,
                   si0, si1, sg0, sg1, so0, so1):
    wid = lax.axis_index("s") * _NC + lax.axis_index("c")
    base = wid * _BPW

    idx_v = (idx0, idx1)
    rows_v = (rows0, rows1)
    si = (si0, si1)
    sg = (sg0, sg1)
    so = (so0, so1)

    def idx_start(i):
        off = base + i * _CHUNK
        pltpu.async_copy(idx_hbm.at[pl.ds(off, _CHUNK)], idx_v[i % 2],
                         si[i % 2])

    def out_start(i, s):
        b_off = (base + i * _CHUNK) // HIST

        def row_body(r, carry):
            for dt in range(4):
                pltpu.async_copy(
                    rows_v[s].at[pl.ds(r * HIST, HIST), pl.ds(dt * 8, 8)],
                    out_hbm.at[b_off + r].at[:, pl.ds(dt * 8, 8)], so[s])
            return carry

        lax.fori_loop(0, _NROWS, row_body, 0)

    def out_wait(i, s):
        b_off = (base + i * _CHUNK) // HIST

        def row_body(r, carry):
            for dt in range(4):
                pltpu.make_async_copy(
                    rows_v[s].at[pl.ds(r * HIST, HIST), pl.ds(dt * 8, 8)],
                    out_hbm.at[b_off + r].at[:, pl.ds(dt * 8, 8)],
                    so[s]).wait()
            return carry

        lax.fori_loop(0, _NROWS, row_body, 0)

    idx_start(0)
    idx_start(1)

    for i in range(_NCHUNK):
        s = i % 2
        pltpu.make_async_copy(
            idx_hbm.at[pl.ds(base + i * _CHUNK, _CHUNK)], idx_v[s], si[s]
        ).wait()
        if i >= 2:
            out_wait(i - 2, s)
        pltpu.async_copy(table_hbm.at[idx_v[s]], rows_v[s], sg[s]).wait()
        out_start(i, s)
        if i + 2 < _NCHUNK:
            idx_start(i + 2)

    for i in (_NCHUNK - 2, _NCHUNK - 1):
        out_wait(i, i % 2)


@jax.jit
def _embed_lookup(idx_flat, table):
    mesh = plsc.VectorSubcoreMesh(core_axis_name="c", subcore_axis_name="s")
    kfn = functools.partial(
        pl.kernel,
        mesh=mesh,
        out_type=jax.ShapeDtypeStruct((BATCH, HIST, LATENT_DIM), jnp.float32),
        scratch_types=[
            pltpu.VMEM((_CHUNK,), jnp.int32),
            pltpu.VMEM((_CHUNK,), jnp.int32),
            pltpu.VMEM((_CHUNK, LATENT_DIM), jnp.float32),
            pltpu.VMEM((_CHUNK, LATENT_DIM), jnp.float32),
            pltpu.SemaphoreType.DMA,
            pltpu.SemaphoreType.DMA,
            pltpu.SemaphoreType.DMA,
            pltpu.SemaphoreType.DMA,
            pltpu.SemaphoreType.DMA,
            pltpu.SemaphoreType.DMA,
        ],
        compiler_params=pltpu.CompilerParams(use_tc_tiling_on_sc=False),
    )(_gather_kernel)
    return kfn(table, idx_flat)


def kernel(input, action_embedding):
    idx_flat = input.astype(jnp.int32).reshape(-1)
    return _embed_lookup(idx_flat, action_embedding)


# speedup vs baseline: 1.5585x; 1.5585x over previous
import functools

import jax
import jax.numpy as jnp
from jax import lax
from jax.experimental import pallas as pl
from jax.experimental.pallas import tpu as pltpu
from jax.experimental.pallas import tpu_sc as plsc

NUM_ACTIONS = 1000000
LATENT_DIM = 32
BATCH = 16384
HIST = 50

_NC = 2
_NS = 16
_NW = _NC * _NS

_NB = BATCH // _NW


def _gather_kernel(table_hbm, idxT_hbm, out_hbm,
                   idx0, idx1, rows0, rows1,
                   si0, si1, sg0, sg1, so0, so1):
    wid = lax.axis_index("s") * _NC + lax.axis_index("c")
    b0 = wid * _NB

    idx_v = (idx0, idx1)
    rows_v = (rows0, rows1)
    si = (si0, si1)
    sg = (sg0, sg1)
    so = (so0, so1)

    def idx_start(h, s):
        pltpu.async_copy(idxT_hbm.at[h, pl.ds(b0, _NB)], idx_v[s], si[s])

    def idx_wait(h, s):
        pltpu.make_async_copy(
            idxT_hbm.at[h, pl.ds(b0, _NB)], idx_v[s], si[s]).wait()

    def gather_start(s):
        pltpu.async_copy(table_hbm.at[idx_v[s]], rows_v[s], sg[s])

    def gather_wait(s):
        pltpu.make_async_copy(
            table_hbm.at[idx_v[s]], rows_v[s], sg[s]).wait()

    def out_start(h, s):
        pltpu.async_copy(rows_v[s], out_hbm.at[h, pl.ds(b0, _NB)], so[s])

    def out_wait(h, s):
        pltpu.make_async_copy(
            rows_v[s], out_hbm.at[h, pl.ds(b0, _NB)], so[s]).wait()

    def step(h, s, wait_so, next_gather, next_idx):
        gather_wait(s)
        if wait_so:
            out_wait(h - 2, s)
        if next_gather:
            idx_wait(h + 1, 1 - s)
            gather_start(1 - s)
        out_start(h, s)
        if next_idx:
            idx_start(h + 2, s)

    idx_start(0, 0)
    idx_wait(0, 0)
    gather_start(0)
    idx_start(1, 1)
    step(0, 0, False, True, True)
    step(1, 1, False, True, True)

    def body(j, carry):
        h = 2 * j
        step(h, 0, True, True, True)
        step(h + 1, 1, True, True, True)
        return carry

    lax.fori_loop(1, HIST // 2 - 1, body, 0)

    step(HIST - 2, 0, True, True, False)
    step(HIST - 1, 1, True, False, False)
    out_wait(HIST - 2, 0)
    out_wait(HIST - 1, 1)


@jax.jit
def _embed_lookup(idxT, table):
    mesh = plsc.VectorSubcoreMesh(core_axis_name="c", subcore_axis_name="s")
    kfn = functools.partial(
        pl.kernel,
        mesh=mesh,
        out_type=jax.ShapeDtypeStruct((HIST, BATCH, LATENT_DIM), jnp.float32),
        scratch_types=[
            pltpu.VMEM((_NB,), jnp.int32),
            pltpu.VMEM((_NB,), jnp.int32),
            pltpu.VMEM((_NB, LATENT_DIM), jnp.float32),
            pltpu.VMEM((_NB, LATENT_DIM), jnp.float32),
            pltpu.SemaphoreType.DMA,
            pltpu.SemaphoreType.DMA,
            pltpu.SemaphoreType.DMA,
            pltpu.SemaphoreType.DMA,
            pltpu.SemaphoreType.DMA,
            pltpu.SemaphoreType.DMA,
        ],
        compiler_params=pltpu.CompilerParams(use_tc_tiling_on_sc=False),
    )(_gather_kernel)
    return kfn(table, idxT)


def kernel(input, action_embedding):
    idxT = input.astype(jnp.int32).T
    out3 = _embed_lookup(idxT, action_embedding)
    return out3.transpose(1, 0, 2)
